# baseline (speedup 1.0000x reference)
import functools

import jax
import jax.numpy as jnp
from jax import lax
from jax.experimental import pallas as pl
from jax.experimental.pallas import tpu as pltpu
from jax.experimental.pallas import tpu_sc as plsc

N_NODES = 10000
N_EDGES = 320000
NODE_DIM = 128
Z_ROWS = 256

NC = 2
NS = 16
L = 16

NODES_PAD = 10240
NODES_PER_TILE = NODES_PAD // (NC * NS)
SUB = 40
N_SUB = NODES_PER_TILE // SUB
EDGES_PER_TILE = N_EDGES // NS
UNROLL = 10


def _body(x_hbm, edges_hbm, zin_hbm, zout_hbm, out_hbm,
          ebuf0, ebuf1, hist_in, hist_out, shared, cbuf,
          idx_in, idx_out, xb0, xb1, zib0, zib1, zob0, zob1,
          sem_e0, sem_e1, sem_c,
          sem_x0, sem_x1, sem_zi0, sem_zi1, sem_zo0, sem_zo1,
          sem_st0, sem_st1):
    c = lax.axis_index("c")
    s = lax.axis_index("s")

    zeros = jnp.zeros((L,), jnp.int32)
    ones = jnp.ones((L,), jnp.int32)

    ebase = s * EDGES_PER_TILE
    cp_e0 = pltpu.async_copy(
        edges_hbm.at[pl.ds(ebase, EDGES_PER_TILE)], ebuf0, sem_e0)
    cp_e1 = pltpu.async_copy(
        edges_hbm.at[pl.ds(N_EDGES + ebase, EDGES_PER_TILE)], ebuf1, sem_e1)

    with jax.named_scope("p1_zero"):
        def zero_body(i, _):
            for u in range(8):
                hist_in[pl.ds((i * 8 + u) * L, L)] = zeros
                hist_out[pl.ds((i * 8 + u) * L, L)] = zeros
            return _

        lax.fori_loop(0, NODES_PAD // (8 * L), zero_body, None)
    with jax.named_scope("p1_ewait"):
        cp_e0.wait()
        cp_e1.wait()

    def edge_body(i, _):
        offs = [(i * UNROLL + u) * L for u in range(UNROLL)]
        srcs = [ebuf0[pl.ds(o, L)] for o in offs]
        dsts = [ebuf1[pl.ds(o, L)] for o in offs]
        for u in range(UNROLL):
            plsc.addupdate_scatter(hist_out, [srcs[u]], ones)
            plsc.addupdate_scatter(hist_in, [dsts[u]], ones)
        return _

    with jax.named_scope("p1_scatter"):
        lax.fori_loop(0, EDGES_PER_TILE // (L * UNROLL), edge_body, None)

    with jax.named_scope("c_publish"):
        pltpu.sync_copy(hist_in, shared.at[pl.ds(s * NODES_PAD, NODES_PAD)])
        pltpu.sync_copy(
            hist_out, shared.at[pl.ds((NS + s) * NODES_PAD, NODES_PAD)])
    with jax.named_scope("c_barrier"):
        plsc.subcore_barrier()

    w = c * NS + s
    gbase = w * NODES_PER_TILE
    zmax = jnp.full((L,), Z_ROWS - 1, jnp.int32)

    with jax.named_scope("c_read"):
        cps = []
        for which in range(2):
            for r in range(NS):
                cps.append(pltpu.async_copy(
                    shared.at[pl.ds((which * NS + r) * NODES_PAD + gbase,
                                    NODES_PER_TILE)],
                    cbuf.at[pl.ds((which * NS + r) * NODES_PER_TILE,
                                  NODES_PER_TILE)],
                    sem_c))
        for cp in cps:
            cp.wait()

    def combine(which, idx_ref):
        def comb_body(j, _):
            base = which * NS * NODES_PER_TILE
            acc = cbuf[pl.ds(base + j * L, L)]
            for r in range(1, NS):
                acc = acc + cbuf[pl.ds(base + r * NODES_PER_TILE + j * L, L)]
            idx_ref[pl.ds(j * L, L)] = jnp.minimum(acc, zmax)
            return _

        lax.fori_loop(0, NODES_PER_TILE // L, comb_body, None)

    with jax.named_scope("c_sum"):
        combine(0, idx_in)
        combine(1, idx_out)

    xb = (xb0, xb1)
    zib = (zib0, zib1)
    zob = (zob0, zob1)
    sem_x = (sem_x0, sem_x1)
    sem_zi = (sem_zi0, sem_zi1)
    sem_zo = (sem_zo0, sem_zo1)
    sem_st = (sem_st0, sem_st1)

    def issue(k):
        b = k % 2
        nbase = gbase + k * SUB

        @pl.when(nbase < N_NODES)
        def _():
            if k >= 2:
                pltpu.make_async_copy(
                    xb[b], out_hbm.at[pl.ds(gbase + (k - 2) * SUB, SUB)],
                    sem_st[b]).wait()
            pltpu.async_copy(x_hbm.at[pl.ds(nbase, SUB)], xb[b], sem_x[b])
            pltpu.async_copy(
                zin_hbm.at[idx_in.at[pl.ds(k * SUB, SUB)]], zib[b], sem_zi[b])
            pltpu.async_copy(
                zout_hbm.at[idx_out.at[pl.ds(k * SUB, SUB)]], zob[b],
                sem_zo[b])

    def process(k):
        b = k % 2
        nbase = gbase + k * SUB

        @pl.when(nbase < N_NODES)
        def _():
            with jax.named_scope("p2_waitx"):
                pltpu.make_async_copy(
                    x_hbm.at[pl.ds(nbase, SUB)], xb[b], sem_x[b]).wait()
            with jax.named_scope("p2_waitz"):
                pltpu.make_async_copy(
                    zin_hbm.at[idx_in.at[pl.ds(k * SUB, SUB)]], zib[b],
                    sem_zi[b]).wait()
                pltpu.make_async_copy(
                    zout_hbm.at[idx_out.at[pl.ds(k * SUB, SUB)]], zob[b],
                    sem_zo[b]).wait()

            def add_body(r, _):
                for cc in range(NODE_DIM // L):
                    sl = pl.ds(cc * L, L)
                    xb[b][r, sl] = xb[b][r, sl] + zib[b][r, sl] + zob[b][r, sl]
                return _

            with jax.named_scope("p2_add"):
                lax.fori_loop(0, SUB, add_body, None)
            pltpu.async_copy(xb[b], out_hbm.at[pl.ds(nbase, SUB)], sem_st[b])

    with jax.named_scope("p2"):
        issue(0)
        for k in range(N_SUB):
            if k + 1 < N_SUB:
                issue(k + 1)
            process(k)

    for k in (N_SUB - 2, N_SUB - 1):
        b = k % 2
        nbase = gbase + k * SUB

        @pl.when(nbase < N_NODES)
        def _():
            pltpu.make_async_copy(
                xb[b], out_hbm.at[pl.ds(nbase, SUB)], sem_st[b]).wait()


@jax.jit
def _centrality(x, edge_index, z_in, z_out):
    mesh = plsc.VectorSubcoreMesh(core_axis_name="c", subcore_axis_name="s")
    run = functools.partial(
        pl.kernel,
        out_type=jax.ShapeDtypeStruct((N_NODES, NODE_DIM), jnp.float32),
        mesh=mesh,
        compiler_params=pltpu.CompilerParams(needs_layout_passes=False),
        scratch_types=[
            pltpu.VMEM((EDGES_PER_TILE,), jnp.int32),
            pltpu.VMEM((EDGES_PER_TILE,), jnp.int32),
            pltpu.VMEM((NODES_PAD,), jnp.int32),
            pltpu.VMEM((NODES_PAD,), jnp.int32),
            pltpu.VMEM_SHARED((2 * NS * NODES_PAD,), jnp.int32),
            pltpu.VMEM((2 * NS * NODES_PER_TILE,), jnp.int32),
            pltpu.VMEM((NODES_PER_TILE,), jnp.int32),
            pltpu.VMEM((NODES_PER_TILE,), jnp.int32),
            pltpu.VMEM((SUB, NODE_DIM), jnp.float32),
            pltpu.VMEM((SUB, NODE_DIM), jnp.float32),
            pltpu.VMEM((SUB, NODE_DIM), jnp.float32),
            pltpu.VMEM((SUB, NODE_DIM), jnp.float32),
            pltpu.VMEM((SUB, NODE_DIM), jnp.float32),
            pltpu.VMEM((SUB, NODE_DIM), jnp.float32),
        ] + [pltpu.SemaphoreType.DMA] * 11,
    )(_body)
    return run(x, edge_index, z_in, z_out)


def kernel(x, edge_index, z_in, z_out):
    edges_flat = edge_index.astype(jnp.int32).reshape(-1)
    return _centrality(x, edges_flat, z_in, z_out)


# speedup vs baseline: 1.7423x; 1.7423x over previous
import functools

import jax
import jax.numpy as jnp
from jax import lax
from jax.experimental import pallas as pl
from jax.experimental.pallas import tpu as pltpu
from jax.experimental.pallas import tpu_sc as plsc

N_NODES = 10000
N_EDGES = 320000
NODE_DIM = 128
Z_ROWS = 256

NC = 2
NS = 16
L = 16

NODES_PAD = 10240
NODES_PER_TILE = NODES_PAD // (NC * NS)
SUB = 40
N_SUB = NODES_PER_TILE // SUB
EDGES_PER_TILE = N_EDGES // NS
UNROLL = 10


def _body(x_hbm, edges_hbm, zin_hbm, zout_hbm, out_hbm,
          ebuf0, ebuf1, hist_in, hist_out, shared, zsh, cbuf,
          idx_in, idx_out, xb0, xb1, zib0, zib1, zob0, zob1,
          sem_e0, sem_e1, sem_c, sem_z,
          sem_x0, sem_x1, sem_zi0, sem_zi1, sem_zo0, sem_zo1,
          sem_st0, sem_st1):
    c = lax.axis_index("c")
    s = lax.axis_index("s")

    zeros = jnp.zeros((L,), jnp.int32)
    ones = jnp.ones((L,), jnp.int32)

    @pl.when(s == 0)
    def _():
        cz0 = pltpu.async_copy(zin_hbm, zsh.at[pl.ds(0, Z_ROWS)], sem_z)
        cz1 = pltpu.async_copy(zout_hbm, zsh.at[pl.ds(Z_ROWS, Z_ROWS)], sem_z)
        cz0.wait()
        cz1.wait()

    ebase = s * EDGES_PER_TILE
    cp_e0 = pltpu.async_copy(
        edges_hbm.at[pl.ds(ebase, EDGES_PER_TILE)], ebuf0, sem_e0)
    cp_e1 = pltpu.async_copy(
        edges_hbm.at[pl.ds(N_EDGES + ebase, EDGES_PER_TILE)], ebuf1, sem_e1)

    with jax.named_scope("p1_zero"):
        def zero_body(i, _):
            for u in range(8):
                hist_in[pl.ds((i * 8 + u) * L, L)] = zeros
                hist_out[pl.ds((i * 8 + u) * L, L)] = zeros
            return _

        lax.fori_loop(0, NODES_PAD // (8 * L), zero_body, None)
    with jax.named_scope("p1_ewait"):
        cp_e0.wait()
        cp_e1.wait()

    def edge_body(i, _):
        offs = [(i * UNROLL + u) * L for u in range(UNROLL)]
        srcs = [ebuf0[pl.ds(o, L)] for o in offs]
        dsts = [ebuf1[pl.ds(o, L)] for o in offs]
        for u in range(UNROLL):
            plsc.addupdate_scatter(hist_out, [srcs[u]], ones)
            plsc.addupdate_scatter(hist_in, [dsts[u]], ones)
        return _

    with jax.named_scope("p1_scatter"):
        lax.fori_loop(0, EDGES_PER_TILE // (L * UNROLL), edge_body, None)

    with jax.named_scope("c_publish"):
        pltpu.sync_copy(hist_in, shared.at[pl.ds(s * NODES_PAD, NODES_PAD)])
        pltpu.sync_copy(
            hist_out, shared.at[pl.ds((NS + s) * NODES_PAD, NODES_PAD)])
    with jax.named_scope("c_barrier"):
        plsc.subcore_barrier()

    w = c * NS + s
    gbase = w * NODES_PER_TILE
    zmax = jnp.full((L,), Z_ROWS - 1, jnp.int32)

    with jax.named_scope("c_read"):
        cps = []
        for which in range(2):
            for r in range(NS):
                cps.append(pltpu.async_copy(
                    shared.at[pl.ds((which * NS + r) * NODES_PAD + gbase,
                                    NODES_PER_TILE)],
                    cbuf.at[pl.ds((which * NS + r) * NODES_PER_TILE,
                                  NODES_PER_TILE)],
                    sem_c))
        for cp in cps:
            cp.wait()

    def combine(which, idx_ref, row_off):
        def comb_body(j, _):
            base = which * NS * NODES_PER_TILE
            acc = cbuf[pl.ds(base + j * L, L)]
            for r in range(1, NS):
                acc = acc + cbuf[pl.ds(base + r * NODES_PER_TILE + j * L, L)]
            idx_ref[pl.ds(j * L, L)] = jnp.minimum(acc, zmax) + row_off
            return _

        lax.fori_loop(0, NODES_PER_TILE // L, comb_body, None)

    with jax.named_scope("c_sum"):
        combine(0, idx_in, 0)
        combine(1, idx_out, Z_ROWS)

    xb = (xb0, xb1)
    zib = (zib0, zib1)
    zob = (zob0, zob1)
    sem_x = (sem_x0, sem_x1)
    sem_zi = (sem_zi0, sem_zi1)
    sem_zo = (sem_zo0, sem_zo1)
    sem_st = (sem_st0, sem_st1)

    def issue(k):
        b = k % 2
        nbase = gbase + k * SUB

        @pl.when(nbase < N_NODES)
        def _():
            if k >= 2:
                pltpu.make_async_copy(
                    xb[b], out_hbm.at[pl.ds(gbase + (k - 2) * SUB, SUB)],
                    sem_st[b]).wait()
            pltpu.async_copy(x_hbm.at[pl.ds(nbase, SUB)], xb[b], sem_x[b])
            pltpu.async_copy(
                zsh.at[idx_in.at[pl.ds(k * SUB, SUB)]], zib[b], sem_zi[b])
            pltpu.async_copy(
                zsh.at[idx_out.at[pl.ds(k * SUB, SUB)]], zob[b],
                sem_zo[b])

    def process(k):
        b = k % 2
        nbase = gbase + k * SUB

        @pl.when(nbase < N_NODES)
        def _():
            with jax.named_scope("p2_waitx"):
                pltpu.make_async_copy(
                    x_hbm.at[pl.ds(nbase, SUB)], xb[b], sem_x[b]).wait()
            with jax.named_scope("p2_waitz"):
                pltpu.make_async_copy(
                    zsh.at[idx_in.at[pl.ds(k * SUB, SUB)]], zib[b],
                    sem_zi[b]).wait()
                pltpu.make_async_copy(
                    zsh.at[idx_out.at[pl.ds(k * SUB, SUB)]], zob[b],
                    sem_zo[b]).wait()

            def add_body(r, _):
                for cc in range(NODE_DIM // L):
                    sl = pl.ds(cc * L, L)
                    xb[b][r, sl] = xb[b][r, sl] + zib[b][r, sl] + zob[b][r, sl]
                return _

            with jax.named_scope("p2_add"):
                lax.fori_loop(0, SUB, add_body, None)
            pltpu.async_copy(xb[b], out_hbm.at[pl.ds(nbase, SUB)], sem_st[b])

    with jax.named_scope("p2"):
        issue(0)
        for k in range(N_SUB):
            if k + 1 < N_SUB:
                issue(k + 1)
            process(k)

    for k in (N_SUB - 2, N_SUB - 1):
        b = k % 2
        nbase = gbase + k * SUB

        @pl.when(nbase < N_NODES)
        def _():
            pltpu.make_async_copy(
                xb[b], out_hbm.at[pl.ds(nbase, SUB)], sem_st[b]).wait()


@jax.jit
def _centrality(x, edge_index, z_in, z_out):
    mesh = plsc.VectorSubcoreMesh(core_axis_name="c", subcore_axis_name="s")
    run = functools.partial(
        pl.kernel,
        out_type=jax.ShapeDtypeStruct((N_NODES, NODE_DIM), jnp.float32),
        mesh=mesh,
        compiler_params=pltpu.CompilerParams(needs_layout_passes=False),
        scratch_types=[
            pltpu.VMEM((EDGES_PER_TILE,), jnp.int32),
            pltpu.VMEM((EDGES_PER_TILE,), jnp.int32),
            pltpu.VMEM((NODES_PAD,), jnp.int32),
            pltpu.VMEM((NODES_PAD,), jnp.int32),
            pltpu.VMEM_SHARED((2 * NS * NODES_PAD,), jnp.int32),
            pltpu.VMEM_SHARED((2 * Z_ROWS, NODE_DIM), jnp.float32),
            pltpu.VMEM((2 * NS * NODES_PER_TILE,), jnp.int32),
            pltpu.VMEM((NODES_PER_TILE,), jnp.int32),
            pltpu.VMEM((NODES_PER_TILE,), jnp.int32),
            pltpu.VMEM((SUB, NODE_DIM), jnp.float32),
            pltpu.VMEM((SUB, NODE_DIM), jnp.float32),
            pltpu.VMEM((SUB, NODE_DIM), jnp.float32),
            pltpu.VMEM((SUB, NODE_DIM), jnp.float32),
            pltpu.VMEM((SUB, NODE_DIM), jnp.float32),
            pltpu.VMEM((SUB, NODE_DIM), jnp.float32),
        ] + [pltpu.SemaphoreType.DMA] * 12,
    )(_body)
    return run(x, edge_index, z_in, z_out)


def kernel(x, edge_index, z_in, z_out):
    edges_flat = edge_index.astype(jnp.int32).reshape(-1)
    return _centrality(x, edges_flat, z_in, z_out)


# speedup vs baseline: 1.7536x; 1.0065x over previous
import functools

import jax
import jax.numpy as jnp
from jax import lax
from jax.experimental import pallas as pl
from jax.experimental.pallas import tpu as pltpu
from jax.experimental.pallas import tpu_sc as plsc

N_NODES = 10000
N_EDGES = 320000
NODE_DIM = 128
Z_ROWS = 256

NC = 2
NS = 16
L = 16

NODES_PAD = 10240
NODES_PER_TILE = NODES_PAD // (NC * NS)
SUB = 40
N_SUB = NODES_PER_TILE // SUB
EDGES_PER_TILE = N_EDGES // NS
UNROLL = 10


def _body(x_hbm, edges_hbm, zin_hbm, zout_hbm, out_hbm,
          ebuf0, ebuf1, hist_in, hist_out, shared, zsh, cbuf,
          idx_in, idx_out, xb0, xb1, zib0, zib1, zob0, zob1,
          sem_e0, sem_e1, sem_c, sem_z,
          sem_x0, sem_x1, sem_zi0, sem_zi1, sem_zo0, sem_zo1,
          sem_st0, sem_st1):
    c = lax.axis_index("c")
    s = lax.axis_index("s")

    zeros = jnp.zeros((L,), jnp.int32)
    ones = jnp.ones((L,), jnp.int32)

    @pl.when(s == 0)
    def _():
        cz0 = pltpu.async_copy(zin_hbm, zsh.at[pl.ds(0, Z_ROWS)], sem_z)
        cz1 = pltpu.async_copy(zout_hbm, zsh.at[pl.ds(Z_ROWS, Z_ROWS)], sem_z)
        cz0.wait()
        cz1.wait()

    ebase = s * EDGES_PER_TILE
    cp_e0 = pltpu.async_copy(
        edges_hbm.at[0, pl.ds(ebase, EDGES_PER_TILE)], ebuf0, sem_e0)
    cp_e1 = pltpu.async_copy(
        edges_hbm.at[1, pl.ds(ebase, EDGES_PER_TILE)], ebuf1, sem_e1)

    with jax.named_scope("p1_zero"):
        def zero_body(i, _):
            for u in range(8):
                hist_in[pl.ds((i * 8 + u) * L, L)] = zeros
                hist_out[pl.ds((i * 8 + u) * L, L)] = zeros
            return _

        lax.fori_loop(0, NODES_PAD // (8 * L), zero_body, None)
    with jax.named_scope("p1_ewait"):
        cp_e0.wait()
        cp_e1.wait()

    def edge_body(i, _):
        offs = [(i * UNROLL + u) * L for u in range(UNROLL)]
        srcs = [ebuf0[pl.ds(o, L)] for o in offs]
        dsts = [ebuf1[pl.ds(o, L)] for o in offs]
        for u in range(UNROLL):
            plsc.addupdate_scatter(hist_out, [srcs[u]], ones)
            plsc.addupdate_scatter(hist_in, [dsts[u]], ones)
        return _

    with jax.named_scope("p1_scatter"):
        lax.fori_loop(0, EDGES_PER_TILE // (L * UNROLL), edge_body, None)

    with jax.named_scope("c_publish"):
        pltpu.sync_copy(hist_in, shared.at[pl.ds(s * NODES_PAD, NODES_PAD)])
        pltpu.sync_copy(
            hist_out, shared.at[pl.ds((NS + s) * NODES_PAD, NODES_PAD)])
    with jax.named_scope("c_barrier"):
        plsc.subcore_barrier()

    w = c * NS + s
    gbase = w * NODES_PER_TILE
    zmax = jnp.full((L,), Z_ROWS - 1, jnp.int32)

    with jax.named_scope("c_read"):
        cps = []
        for which in range(2):
            for r in range(NS):
                cps.append(pltpu.async_copy(
                    shared.at[pl.ds((which * NS + r) * NODES_PAD + gbase,
                                    NODES_PER_TILE)],
                    cbuf.at[pl.ds((which * NS + r) * NODES_PER_TILE,
                                  NODES_PER_TILE)],
                    sem_c))
        for cp in cps:
            cp.wait()

    def combine(which, idx_ref, row_off):
        def comb_body(j, _):
            base = which * NS * NODES_PER_TILE
            acc = cbuf[pl.ds(base + j * L, L)]
            for r in range(1, NS):
                acc = acc + cbuf[pl.ds(base + r * NODES_PER_TILE + j * L, L)]
            idx_ref[pl.ds(j * L, L)] = jnp.minimum(acc, zmax) + row_off
            return _

        lax.fori_loop(0, NODES_PER_TILE // L, comb_body, None)

    with jax.named_scope("c_sum"):
        combine(0, idx_in, 0)
        combine(1, idx_out, Z_ROWS)

    xb = (xb0, xb1)
    zib = (zib0, zib1)
    zob = (zob0, zob1)
    sem_x = (sem_x0, sem_x1)
    sem_zi = (sem_zi0, sem_zi1)
    sem_zo = (sem_zo0, sem_zo1)
    sem_st = (sem_st0, sem_st1)

    def issue(k):
        b = k % 2
        nbase = gbase + k * SUB

        @pl.when(nbase < N_NODES)
        def _():
            if k >= 2:
                pltpu.make_async_copy(
                    xb[b], out_hbm.at[pl.ds(gbase + (k - 2) * SUB, SUB)],
                    sem_st[b]).wait()
            pltpu.async_copy(x_hbm.at[pl.ds(nbase, SUB)], xb[b], sem_x[b])
            pltpu.async_copy(
                zsh.at[idx_in.at[pl.ds(k * SUB, SUB)]], zib[b], sem_zi[b])
            pltpu.async_copy(
                zsh.at[idx_out.at[pl.ds(k * SUB, SUB)]], zob[b],
                sem_zo[b])

    def process(k):
        b = k % 2
        nbase = gbase + k * SUB

        @pl.when(nbase < N_NODES)
        def _():
            with jax.named_scope("p2_waitx"):
                pltpu.make_async_copy(
                    x_hbm.at[pl.ds(nbase, SUB)], xb[b], sem_x[b]).wait()
            with jax.named_scope("p2_waitz"):
                pltpu.make_async_copy(
                    zsh.at[idx_in.at[pl.ds(k * SUB, SUB)]], zib[b],
                    sem_zi[b]).wait()
                pltpu.make_async_copy(
                    zsh.at[idx_out.at[pl.ds(k * SUB, SUB)]], zob[b],
                    sem_zo[b]).wait()

            def add_body(r, _):
                for cc in range(NODE_DIM // L):
                    sl = pl.ds(cc * L, L)
                    xb[b][r, sl] = xb[b][r, sl] + zib[b][r, sl] + zob[b][r, sl]
                return _

            with jax.named_scope("p2_add"):
                lax.fori_loop(0, SUB, add_body, None)
            pltpu.async_copy(xb[b], out_hbm.at[pl.ds(nbase, SUB)], sem_st[b])

    with jax.named_scope("p2"):
        issue(0)
        for k in range(N_SUB):
            if k + 1 < N_SUB:
                issue(k + 1)
            process(k)

    for k in (N_SUB - 2, N_SUB - 1):
        b = k % 2
        nbase = gbase + k * SUB

        @pl.when(nbase < N_NODES)
        def _():
            pltpu.make_async_copy(
                xb[b], out_hbm.at[pl.ds(nbase, SUB)], sem_st[b]).wait()


@jax.jit
def _centrality(x, edge_index, z_in, z_out):
    mesh = plsc.VectorSubcoreMesh(core_axis_name="c", subcore_axis_name="s")
    run = functools.partial(
        pl.kernel,
        out_type=jax.ShapeDtypeStruct((N_NODES, NODE_DIM), jnp.float32),
        mesh=mesh,
        compiler_params=pltpu.CompilerParams(
            needs_layout_passes=False, use_tc_tiling_on_sc=False),
        scratch_types=[
            pltpu.VMEM((EDGES_PER_TILE,), jnp.int32),
            pltpu.VMEM((EDGES_PER_TILE,), jnp.int32),
            pltpu.VMEM((NODES_PAD,), jnp.int32),
            pltpu.VMEM((NODES_PAD,), jnp.int32),
            pltpu.VMEM_SHARED((2 * NS * NODES_PAD,), jnp.int32),
            pltpu.VMEM_SHARED((2 * Z_ROWS, NODE_DIM), jnp.float32),
            pltpu.VMEM((2 * NS * NODES_PER_TILE,), jnp.int32),
            pltpu.VMEM((NODES_PER_TILE,), jnp.int32),
            pltpu.VMEM((NODES_PER_TILE,), jnp.int32),
            pltpu.VMEM((SUB, NODE_DIM), jnp.float32),
            pltpu.VMEM((SUB, NODE_DIM), jnp.float32),
            pltpu.VMEM((SUB, NODE_DIM), jnp.float32),
            pltpu.VMEM((SUB, NODE_DIM), jnp.float32),
            pltpu.VMEM((SUB, NODE_DIM), jnp.float32),
            pltpu.VMEM((SUB, NODE_DIM), jnp.float32),
        ] + [pltpu.SemaphoreType.DMA] * 12,
    )(_body)
    return run(x, edge_index, z_in, z_out)


def kernel(x, edge_index, z_in, z_out):
    return _centrality(x, edge_index.astype(jnp.int32), z_in, z_out)


# speedup vs baseline: 1.8141x; 1.0345x over previous
import functools

import jax
import jax.numpy as jnp
from jax import lax
from jax.experimental import pallas as pl
from jax.experimental.pallas import tpu as pltpu
from jax.experimental.pallas import tpu_sc as plsc

N_NODES = 10000
N_EDGES = 320000
NODE_DIM = 128
Z_ROWS = 256

NC = 2
NS = 16
L = 16

NODES_PAD = 10240
NODES_PER_TILE = NODES_PAD // (NC * NS)
SUB = 40
N_SUB = NODES_PER_TILE // SUB
EBLK = 128
NBLK_BASE = 156
EDGES_BASE = NBLK_BASE * EBLK
UNROLL = 8


def _body(x_hbm, edges_hbm, zin_hbm, zout_hbm, out_hbm,
          ebuf0, ebuf1, hist_in, hist_out, shared, zsh, cbuf,
          idx_in, idx_out, xb0, xb1, zib0, zib1, zob0, zob1,
          sem_e0, sem_e1, sem_c, sem_z,
          sem_x0, sem_x1, sem_zi0, sem_zi1, sem_zo0, sem_zo1,
          sem_st0, sem_st1):
    c = lax.axis_index("c")
    s = lax.axis_index("s")

    zeros = jnp.zeros((L,), jnp.int32)
    ones = jnp.ones((L,), jnp.int32)

    @pl.when(s == 0)
    def _():
        cz0 = pltpu.async_copy(zin_hbm, zsh.at[pl.ds(0, Z_ROWS)], sem_z)
        cz1 = pltpu.async_copy(zout_hbm, zsh.at[pl.ds(Z_ROWS, Z_ROWS)], sem_z)
        cz0.wait()
        cz1.wait()

    ecol0 = (s * NBLK_BASE + jnp.minimum(s, 4)) * EBLK
    has_extra = s < 4
    cp_e0 = pltpu.async_copy(
        edges_hbm.at[:, pl.ds(ecol0, EDGES_BASE)], ebuf0, sem_e0)

    @pl.when(has_extra)
    def _():
        pltpu.async_copy(
            edges_hbm.at[:, pl.ds(ecol0 + EDGES_BASE, EBLK)], ebuf1, sem_e1)

    with jax.named_scope("p1_zero"):
        def zero_body(i, _):
            for u in range(8):
                hist_in[pl.ds((i * 8 + u) * L, L)] = zeros
                hist_out[pl.ds((i * 8 + u) * L, L)] = zeros
            return _

        lax.fori_loop(0, NODES_PAD // (8 * L), zero_body, None)
    with jax.named_scope("p1_ewait"):
        cp_e0.wait()

        @pl.when(has_extra)
        def _():
            pltpu.make_async_copy(
                edges_hbm.at[:, pl.ds(ecol0 + EDGES_BASE, EBLK)], ebuf1,
                sem_e1).wait()

    def scatter_batch(ebuf, offs):
        srcs = [ebuf[0, pl.ds(o, L)] for o in offs]
        dsts = [ebuf[1, pl.ds(o, L)] for o in offs]
        for u in range(len(offs)):
            plsc.addupdate_scatter(hist_out, [srcs[u]], ones)
            plsc.addupdate_scatter(hist_in, [dsts[u]], ones)

    def edge_body(i, _):
        scatter_batch(ebuf0, [(i * UNROLL + u) * L for u in range(UNROLL)])
        return _

    with jax.named_scope("p1_scatter"):
        lax.fori_loop(0, EDGES_BASE // (L * UNROLL), edge_body, None)

        @pl.when(has_extra)
        def _():
            scatter_batch(ebuf1, [u * L for u in range(EBLK // L)])

    with jax.named_scope("c_publish"):
        pltpu.sync_copy(hist_in, shared.at[pl.ds(s * NODES_PAD, NODES_PAD)])
        pltpu.sync_copy(
            hist_out, shared.at[pl.ds((NS + s) * NODES_PAD, NODES_PAD)])
    with jax.named_scope("c_barrier"):
        plsc.subcore_barrier()

    w = c * NS + s
    gbase = w * NODES_PER_TILE
    zmax = jnp.full((L,), Z_ROWS - 1, jnp.int32)

    with jax.named_scope("c_read"):
        cps = []
        for which in range(2):
            for r in range(NS):
                cps.append(pltpu.async_copy(
                    shared.at[pl.ds((which * NS + r) * NODES_PAD + gbase,
                                    NODES_PER_TILE)],
                    cbuf.at[pl.ds((which * NS + r) * NODES_PER_TILE,
                                  NODES_PER_TILE)],
                    sem_c))
        for cp in cps:
            cp.wait()

    def combine(which, idx_ref, row_off):
        def comb_body(j, _):
            base = which * NS * NODES_PER_TILE
            acc = cbuf[pl.ds(base + j * L, L)]
            for r in range(1, NS):
                acc = acc + cbuf[pl.ds(base + r * NODES_PER_TILE + j * L, L)]
            idx_ref[pl.ds(j * L, L)] = jnp.minimum(acc, zmax) + row_off
            return _

        lax.fori_loop(0, NODES_PER_TILE // L, comb_body, None)

    with jax.named_scope("c_sum"):
        combine(0, idx_in, 0)
        combine(1, idx_out, Z_ROWS)

    xb = (xb0, xb1)
    zib = (zib0, zib1)
    zob = (zob0, zob1)
    sem_x = (sem_x0, sem_x1)
    sem_zi = (sem_zi0, sem_zi1)
    sem_zo = (sem_zo0, sem_zo1)
    sem_st = (sem_st0, sem_st1)

    def issue(k):
        b = k % 2
        nbase = gbase + k * SUB

        @pl.when(nbase < N_NODES)
        def _():
            if k >= 2:
                pltpu.make_async_copy(
                    xb[b], out_hbm.at[pl.ds(gbase + (k - 2) * SUB, SUB)],
                    sem_st[b]).wait()
            pltpu.async_copy(x_hbm.at[pl.ds(nbase, SUB)], xb[b], sem_x[b])
            pltpu.async_copy(
                zsh.at[idx_in.at[pl.ds(k * SUB, SUB)]], zib[b], sem_zi[b])
            pltpu.async_copy(
                zsh.at[idx_out.at[pl.ds(k * SUB, SUB)]], zob[b],
                sem_zo[b])

    def process(k):
        b = k % 2
        nbase = gbase + k * SUB

        @pl.when(nbase < N_NODES)
        def _():
            with jax.named_scope("p2_waitx"):
                pltpu.make_async_copy(
                    x_hbm.at[pl.ds(nbase, SUB)], xb[b], sem_x[b]).wait()
            with jax.named_scope("p2_waitz"):
                pltpu.make_async_copy(
                    zsh.at[idx_in.at[pl.ds(k * SUB, SUB)]], zib[b],
                    sem_zi[b]).wait()
                pltpu.make_async_copy(
                    zsh.at[idx_out.at[pl.ds(k * SUB, SUB)]], zob[b],
                    sem_zo[b]).wait()

            def add_body(r, _):
                for cc in range(NODE_DIM // L):
                    sl = pl.ds(cc * L, L)
                    xb[b][r, sl] = xb[b][r, sl] + zib[b][r, sl] + zob[b][r, sl]
                return _

            with jax.named_scope("p2_add"):
                lax.fori_loop(0, SUB, add_body, None)
            pltpu.async_copy(xb[b], out_hbm.at[pl.ds(nbase, SUB)], sem_st[b])

    with jax.named_scope("p2"):
        issue(0)
        for k in range(N_SUB):
            if k + 1 < N_SUB:
                issue(k + 1)
            process(k)

    for k in (N_SUB - 2, N_SUB - 1):
        b = k % 2
        nbase = gbase + k * SUB

        @pl.when(nbase < N_NODES)
        def _():
            pltpu.make_async_copy(
                xb[b], out_hbm.at[pl.ds(nbase, SUB)], sem_st[b]).wait()


@jax.jit
def _centrality(x, edge_index, z_in, z_out):
    mesh = plsc.VectorSubcoreMesh(core_axis_name="c", subcore_axis_name="s")
    run = functools.partial(
        pl.kernel,
        out_type=jax.ShapeDtypeStruct((N_NODES, NODE_DIM), jnp.float32),
        mesh=mesh,
        compiler_params=pltpu.CompilerParams(needs_layout_passes=False),
        scratch_types=[
            pltpu.VMEM((2, EDGES_BASE), jnp.int32),
            pltpu.VMEM((2, EBLK), jnp.int32),
            pltpu.VMEM((NODES_PAD,), jnp.int32),
            pltpu.VMEM((NODES_PAD,), jnp.int32),
            pltpu.VMEM_SHARED((2 * NS * NODES_PAD,), jnp.int32),
            pltpu.VMEM_SHARED((2 * Z_ROWS, NODE_DIM), jnp.float32),
            pltpu.VMEM((2 * NS * NODES_PER_TILE,), jnp.int32),
            pltpu.VMEM((NODES_PER_TILE,), jnp.int32),
            pltpu.VMEM((NODES_PER_TILE,), jnp.int32),
            pltpu.VMEM((SUB, NODE_DIM), jnp.float32),
            pltpu.VMEM((SUB, NODE_DIM), jnp.float32),
            pltpu.VMEM((SUB, NODE_DIM), jnp.float32),
            pltpu.VMEM((SUB, NODE_DIM), jnp.float32),
            pltpu.VMEM((SUB, NODE_DIM), jnp.float32),
            pltpu.VMEM((SUB, NODE_DIM), jnp.float32),
        ] + [pltpu.SemaphoreType.DMA] * 12,
    )(_body)
    return run(x, edge_index, z_in, z_out)


def kernel(x, edge_index, z_in, z_out):
    return _centrality(x, edge_index.astype(jnp.int32), z_in, z_out)


# speedup vs baseline: 1.9630x; 1.0821x over previous
import functools

import jax
import jax.numpy as jnp
from jax import lax
from jax.experimental import pallas as pl
from jax.experimental.pallas import tpu as pltpu
from jax.experimental.pallas import tpu_sc as plsc

N_NODES = 10000
N_EDGES = 320000
NODE_DIM = 128
Z_ROWS = 256

NC = 2
NS = 16
L = 16

NODES_PAD = 10240
NODES_PER_TILE = NODES_PAD // (NC * NS)
SUB = 40
N_SUB = NODES_PER_TILE // SUB
NX = 4
NZ = 3
EBLK = 128
NBLK_BASE = 156
EDGES_BASE = NBLK_BASE * EBLK
ECHUNKS = 4
ECH_BLKS = NBLK_BASE // ECHUNKS
ECH = ECH_BLKS * EBLK
UNROLL = 8


def _body(x_hbm, edges_hbm, zin_hbm, zout_hbm, out_hbm,
          ebuf0a, ebuf0b, ebuf1, hist_in, hist_out, shared, zsh, cbuf,
          idx_in, idx_out, xb, zib, zob,
          sem_e0, sem_e0b, sem_e1, sem_c, sem_z,
          sem_x, sem_zi, sem_zo, sem_st):
    c = lax.axis_index("c")
    s = lax.axis_index("s")

    zeros = jnp.zeros((L,), jnp.int32)
    ones = jnp.ones((L,), jnp.int32)

    w = c * NS + s
    gbase = w * NODES_PER_TILE

    def issue_x(k):
        bx = k % NX
        nbase = gbase + k * SUB

        @pl.when(nbase < N_NODES)
        def _():
            if k >= NX:
                pltpu.make_async_copy(
                    xb[bx], out_hbm.at[pl.ds(gbase + (k - NX) * SUB, SUB)],
                    sem_st[bx]).wait()
            pltpu.async_copy(x_hbm.at[pl.ds(nbase, SUB)], xb[bx], sem_x[bx])

    def issue_z(k):
        bz = k % NZ
        nbase = gbase + k * SUB

        @pl.when(nbase < N_NODES)
        def _():
            pltpu.async_copy(
                zsh.at[idx_in.at[pl.ds(k * SUB, SUB)]], zib[bz], sem_zi[bz])
            pltpu.async_copy(
                zsh.at[idx_out.at[pl.ds(k * SUB, SUB)]], zob[bz], sem_zo[bz])

    @pl.when(s == 0)
    def _():
        pltpu.async_copy(zin_hbm, zsh.at[pl.ds(0, Z_ROWS)], sem_z)
        pltpu.async_copy(zout_hbm, zsh.at[pl.ds(Z_ROWS, Z_ROWS)], sem_z)

    ecol0 = (s * NBLK_BASE + jnp.minimum(s, 4)) * EBLK
    has_extra = s < 4
    eb = (ebuf0a, ebuf0b)
    sem_e = (sem_e0, sem_e0b)

    def issue_e(t):
        pltpu.async_copy(
            edges_hbm.at[:, pl.ds(ecol0 + t * ECH, ECH)], eb[t % 2],
            sem_e[t % 2])

    issue_e(0)
    issue_e(1)

    @pl.when(has_extra)
    def _():
        pltpu.async_copy(
            edges_hbm.at[:, pl.ds(ecol0 + EDGES_BASE, EBLK)], ebuf1, sem_e1)

    for k in range(NX - 2):
        issue_x(k)

    with jax.named_scope("p1_zero"):
        def zero_body(i, _):
            for u in range(8):
                hist_in[pl.ds((i * 8 + u) * L, L)] = zeros
                hist_out[pl.ds((i * 8 + u) * L, L)] = zeros
            return _

        lax.fori_loop(0, NODES_PAD // (8 * L), zero_body, None)

    def scatter_batch(ebuf, offs):
        srcs = [ebuf[0, pl.ds(o, L)] for o in offs]
        dsts = [ebuf[1, pl.ds(o, L)] for o in offs]
        for u in range(len(offs)):
            plsc.addupdate_scatter(hist_out, [srcs[u]], ones)
            plsc.addupdate_scatter(hist_in, [dsts[u]], ones)

    with jax.named_scope("p1_scatter"):
        for t in range(ECHUNKS):
            with jax.named_scope("p1_ewait"):
                pltpu.make_async_copy(
                    edges_hbm.at[:, pl.ds(ecol0 + t * ECH, ECH)], eb[t % 2],
                    sem_e[t % 2]).wait()
            def edge_body(i, _, _eb=eb[t % 2]):
                scatter_batch(
                    _eb, [(i * UNROLL + u) * L for u in range(UNROLL)])
                return _

            lax.fori_loop(0, ECH // (L * UNROLL), edge_body, None)
            if t + 2 < ECHUNKS:
                issue_e(t + 2)

        @pl.when(has_extra)
        def _():
            pltpu.make_async_copy(
                edges_hbm.at[:, pl.ds(ecol0 + EDGES_BASE, EBLK)], ebuf1,
                sem_e1).wait()
            scatter_batch(ebuf1, [u * L for u in range(EBLK // L)])

    @pl.when(s == 0)
    def _():
        pltpu.make_async_copy(
            zout_hbm, zsh.at[pl.ds(Z_ROWS, Z_ROWS)], sem_z).wait()
        pltpu.make_async_copy(
            zin_hbm, zsh.at[pl.ds(0, Z_ROWS)], sem_z).wait()

    with jax.named_scope("c_publish"):
        pltpu.sync_copy(hist_in, shared.at[pl.ds(s * NODES_PAD, NODES_PAD)])
        pltpu.sync_copy(
            hist_out, shared.at[pl.ds((NS + s) * NODES_PAD, NODES_PAD)])
    with jax.named_scope("c_barrier"):
        plsc.subcore_barrier()

    zmax = jnp.full((L,), Z_ROWS - 1, jnp.int32)

    def read_partials(which):
        cps = []
        for r in range(NS):
            cps.append(pltpu.async_copy(
                shared.at[pl.ds((which * NS + r) * NODES_PAD + gbase,
                                NODES_PER_TILE)],
                cbuf.at[pl.ds(r * NODES_PER_TILE, NODES_PER_TILE)],
                sem_c))
        return cps

    def combine(idx_ref, row_off):
        def comb_body(j, _):
            acc = cbuf[pl.ds(j * L, L)]
            for r in range(1, NS):
                acc = acc + cbuf[pl.ds(r * NODES_PER_TILE + j * L, L)]
            idx_ref[pl.ds(j * L, L)] = jnp.minimum(acc, zmax) + row_off
            return _

        lax.fori_loop(0, NODES_PER_TILE // L, comb_body, None)

    with jax.named_scope("c_read"):
        for cp in read_partials(0):
            cp.wait()
    with jax.named_scope("c_sum"):
        combine(idx_in, 0)
    with jax.named_scope("c_read2"):
        for cp in read_partials(1):
            cp.wait()
    with jax.named_scope("c_sum2"):
        combine(idx_out, Z_ROWS)

    with jax.named_scope("p2_zissue"):
        for k in range(NZ - 1):
            issue_z(k)

    def process(k):
        bx = k % NX
        bz = k % NZ
        nbase = gbase + k * SUB

        @pl.when(nbase < N_NODES)
        def _():
            with jax.named_scope("p2_waitx"):
                pltpu.make_async_copy(
                    x_hbm.at[pl.ds(nbase, SUB)], xb[bx], sem_x[bx]).wait()
            with jax.named_scope("p2_waitz"):
                pltpu.make_async_copy(
                    zsh.at[idx_in.at[pl.ds(k * SUB, SUB)]], zib[bz],
                    sem_zi[bz]).wait()
                pltpu.make_async_copy(
                    zsh.at[idx_out.at[pl.ds(k * SUB, SUB)]], zob[bz],
                    sem_zo[bz]).wait()

            def add_body(r, _):
                for cc in range(NODE_DIM // L):
                    sl = pl.ds(cc * L, L)
                    xb[bx][r, sl] = (
                        xb[bx][r, sl] + zib[bz][r, sl] + zob[bz][r, sl])
                return _

            with jax.named_scope("p2_add"):
                lax.fori_loop(0, SUB, add_body, None)
            pltpu.async_copy(xb[bx], out_hbm.at[pl.ds(nbase, SUB)],
                             sem_st[bx])

    with jax.named_scope("p2"):
        for k in range(N_SUB):
            if k + NZ - 1 < N_SUB:
                issue_z(k + NZ - 1)
            if k + NX - 2 < N_SUB:
                issue_x(k + NX - 2)
            process(k)

    for k in range(max(0, N_SUB - NX), N_SUB):
        bx = k % NX
        nbase = gbase + k * SUB

        @pl.when(nbase < N_NODES)
        def _():
            pltpu.make_async_copy(
                xb[bx], out_hbm.at[pl.ds(nbase, SUB)], sem_st[bx]).wait()


def _flat_body(x_hbm, edges_hbm, zin_hbm, zout_hbm, out_hbm, *scratch):
    (ebuf0a, ebuf0b, ebuf1, hist_in, hist_out, shared, zsh, cbuf,
     idx_in, idx_out,
     xb0, xb1, xb2, xb3, zib0, zib1, zib2, zob0, zob1, zob2,
     sem_e0, sem_e0b, sem_e1, sem_c, sem_z,
     sx0, sx1, sx2, sx3, szi0, szi1, szi2, szo0, szo1, szo2,
     st0, st1, st2, st3) = scratch
    _body(x_hbm, edges_hbm, zin_hbm, zout_hbm, out_hbm,
          ebuf0a, ebuf0b, ebuf1, hist_in, hist_out, shared, zsh, cbuf,
          idx_in, idx_out,
          (xb0, xb1, xb2, xb3), (zib0, zib1, zib2), (zob0, zob1, zob2),
          sem_e0, sem_e0b, sem_e1, sem_c, sem_z,
          (sx0, sx1, sx2, sx3), (szi0, szi1, szi2), (szo0, szo1, szo2),
          (st0, st1, st2, st3))


@jax.jit
def _centrality(x, edge_index, z_in, z_out):
    mesh = plsc.VectorSubcoreMesh(core_axis_name="c", subcore_axis_name="s")
    run = functools.partial(
        pl.kernel,
        out_type=jax.ShapeDtypeStruct((N_NODES, NODE_DIM), jnp.float32),
        mesh=mesh,
        compiler_params=pltpu.CompilerParams(needs_layout_passes=False),
        scratch_types=[
            pltpu.VMEM((2, ECH), jnp.int32),
            pltpu.VMEM((2, ECH), jnp.int32),
            pltpu.VMEM((2, EBLK), jnp.int32),
            pltpu.VMEM((NODES_PAD,), jnp.int32),
            pltpu.VMEM((NODES_PAD,), jnp.int32),
            pltpu.VMEM_SHARED((2 * NS * NODES_PAD,), jnp.int32),
            pltpu.VMEM_SHARED((2 * Z_ROWS, NODE_DIM), jnp.float32),
            pltpu.VMEM((NS * NODES_PER_TILE,), jnp.int32),
            pltpu.VMEM((NODES_PER_TILE,), jnp.int32),
            pltpu.VMEM((NODES_PER_TILE,), jnp.int32),
        ]
        + [pltpu.VMEM((SUB, NODE_DIM), jnp.float32)] * (NX + 2 * NZ)
        + [pltpu.SemaphoreType.DMA] * (5 + NX + 2 * NZ + NX),
    )(_flat_body)
    return run(x, edge_index, z_in, z_out)


def kernel(x, edge_index, z_in, z_out):
    return _centrality(x, edge_index.astype(jnp.int32), z_in, z_out)


# speedup vs baseline: 1.9682x; 1.0026x over previous
import functools

import jax
import jax.numpy as jnp
from jax import lax
from jax.experimental import pallas as pl
from jax.experimental.pallas import tpu as pltpu
from jax.experimental.pallas import tpu_sc as plsc

N_NODES = 10000
N_EDGES = 320000
NODE_DIM = 128
Z_ROWS = 256

NC = 2
NS = 16
L = 16

NODES_PAD = 10240
NODES_PER_TILE = NODES_PAD // (NC * NS)
SUB = 40
N_SUB = NODES_PER_TILE // SUB
NX = 4
NZ = 3
EBLK = 128
NBLK_BASE = 156
EDGES_BASE = NBLK_BASE * EBLK
ECHUNKS = 4
ECH_BLKS = NBLK_BASE // ECHUNKS
ECH = ECH_BLKS * EBLK
UNROLL = 8


def _body(x_hbm, edges_hbm, zin_hbm, zout_hbm, out_hbm,
          ebuf0a, ebuf0b, ebuf1, hist_in, hist_out, shared, zsh, cbuf,
          idx_in, idx_out, xb, zib, zob,
          sem_e0, sem_e0b, sem_e1, sem_c, sem_z,
          sem_x, sem_zi, sem_zo, sem_st):
    c = lax.axis_index("c")
    s = lax.axis_index("s")

    zeros = jnp.zeros((L,), jnp.int32)
    ones = jnp.ones((L,), jnp.int32)

    w = c * NS + s
    gbase = w * NODES_PER_TILE

    def issue_x(k):
        bx = k % NX
        nbase = gbase + k * SUB

        @pl.when(nbase < N_NODES)
        def _():
            if k >= NX:
                pltpu.make_async_copy(
                    xb[bx], out_hbm.at[pl.ds(gbase + (k - NX) * SUB, SUB)],
                    sem_st[bx]).wait()
            pltpu.async_copy(x_hbm.at[pl.ds(nbase, SUB)], xb[bx], sem_x[bx])

    def issue_z(k):
        bz = k % NZ
        nbase = gbase + k * SUB

        @pl.when(nbase < N_NODES)
        def _():
            pltpu.async_copy(
                zsh.at[idx_in.at[pl.ds(k * SUB, SUB)]], zib[bz], sem_zi[bz])
            pltpu.async_copy(
                zsh.at[idx_out.at[pl.ds(k * SUB, SUB)]], zob[bz], sem_zo[bz])

    @pl.when(s == 0)
    def _():
        pltpu.async_copy(zin_hbm, zsh.at[pl.ds(0, Z_ROWS)], sem_z)
        pltpu.async_copy(zout_hbm, zsh.at[pl.ds(Z_ROWS, Z_ROWS)], sem_z)

    ecol0 = (s * NBLK_BASE + jnp.minimum(s, 4)) * EBLK
    has_extra = s < 4
    eb = (ebuf0a, ebuf0b)
    sem_e = (sem_e0, sem_e0b)

    def issue_e(t):
        pltpu.async_copy(
            edges_hbm.at[:, pl.ds(ecol0 + t * ECH, ECH)], eb[t % 2],
            sem_e[t % 2])

    issue_e(0)
    issue_e(1)

    @pl.when(has_extra)
    def _():
        pltpu.async_copy(
            edges_hbm.at[:, pl.ds(ecol0 + EDGES_BASE, EBLK)], ebuf1, sem_e1)

    for k in range(NX - 2):
        issue_x(k)

    with jax.named_scope("p1_zero"):
        def zero_body(i, _):
            for u in range(8):
                hist_in[pl.ds((i * 8 + u) * L, L)] = zeros
                hist_out[pl.ds((i * 8 + u) * L, L)] = zeros
            return _

        lax.fori_loop(0, NODES_PAD // (8 * L), zero_body, None)

    def scatter_batch(ebuf, offs):
        srcs = [ebuf[0, pl.ds(o, L)] for o in offs]
        dsts = [ebuf[1, pl.ds(o, L)] for o in offs]
        for u in range(len(offs)):
            plsc.addupdate_scatter(hist_out, [srcs[u]], ones)
            plsc.addupdate_scatter(hist_in, [dsts[u]], ones)

    with jax.named_scope("p1_scatter"):
        for t in range(ECHUNKS):
            with jax.named_scope("p1_ewait"):
                pltpu.make_async_copy(
                    edges_hbm.at[:, pl.ds(ecol0 + t * ECH, ECH)], eb[t % 2],
                    sem_e[t % 2]).wait()
            def edge_body(i, _, _eb=eb[t % 2]):
                scatter_batch(
                    _eb, [(i * UNROLL + u) * L for u in range(UNROLL)])
                return _

            lax.fori_loop(0, ECH // (L * UNROLL), edge_body, None)
            if t + 2 < ECHUNKS:
                issue_e(t + 2)

        @pl.when(has_extra)
        def _():
            pltpu.make_async_copy(
                edges_hbm.at[:, pl.ds(ecol0 + EDGES_BASE, EBLK)], ebuf1,
                sem_e1).wait()
            scatter_batch(ebuf1, [u * L for u in range(EBLK // L)])

    @pl.when(s == 0)
    def _():
        pltpu.make_async_copy(
            zout_hbm, zsh.at[pl.ds(Z_ROWS, Z_ROWS)], sem_z).wait()
        pltpu.make_async_copy(
            zin_hbm, zsh.at[pl.ds(0, Z_ROWS)], sem_z).wait()

    with jax.named_scope("c_publish"):
        pltpu.sync_copy(hist_in, shared.at[pl.ds(s * NODES_PAD, NODES_PAD)])
        pltpu.sync_copy(
            hist_out, shared.at[pl.ds((NS + s) * NODES_PAD, NODES_PAD)])
    with jax.named_scope("c_barrier"):
        plsc.subcore_barrier()

    zmax = jnp.full((L,), Z_ROWS - 1, jnp.int32)

    def read_partials(which):
        cps = []
        for r in range(NS):
            cps.append(pltpu.async_copy(
                shared.at[pl.ds((which * NS + r) * NODES_PAD + gbase,
                                NODES_PER_TILE)],
                cbuf.at[pl.ds(r * NODES_PER_TILE, NODES_PER_TILE)],
                sem_c))
        return cps

    def combine(idx_ref, row_off):
        def comb_body(j, _):
            acc = cbuf[pl.ds(j * L, L)]
            for r in range(1, NS):
                acc = acc + cbuf[pl.ds(r * NODES_PER_TILE + j * L, L)]
            idx_ref[pl.ds(j * L, L)] = jnp.minimum(acc, zmax) + row_off
            return _

        lax.fori_loop(0, NODES_PER_TILE // L, comb_body, None)

    with jax.named_scope("c_read"):
        for cp in read_partials(0):
            cp.wait()
    with jax.named_scope("c_sum"):
        combine(idx_in, 0)
    with jax.named_scope("c_read2"):
        for cp in read_partials(1):
            cp.wait()
    with jax.named_scope("c_sum2"):
        combine(idx_out, Z_ROWS)

    with jax.named_scope("p2_zissue"):
        for k in range(NZ - 1):
            issue_z(k)

    def process(k):
        bx = k % NX
        bz = k % NZ
        nbase = gbase + k * SUB

        @pl.when(nbase < N_NODES)
        def _():
            with jax.named_scope("p2_waitx"):
                pltpu.make_async_copy(
                    x_hbm.at[pl.ds(nbase, SUB)], xb[bx], sem_x[bx]).wait()
            with jax.named_scope("p2_waitz"):
                pltpu.make_async_copy(
                    zsh.at[idx_in.at[pl.ds(k * SUB, SUB)]], zib[bz],
                    sem_zi[bz]).wait()
                pltpu.make_async_copy(
                    zsh.at[idx_out.at[pl.ds(k * SUB, SUB)]], zob[bz],
                    sem_zo[bz]).wait()

            def add_body(r, _):
                for cc in range(NODE_DIM // L):
                    sl = pl.ds(cc * L, L)
                    xb[bx][r, sl] = (
                        xb[bx][r, sl] + zib[bz][r, sl] + zob[bz][r, sl])
                return _

            with jax.named_scope("p2_add"):
                lax.fori_loop(0, SUB, add_body, None)
            pltpu.async_copy(xb[bx], out_hbm.at[pl.ds(nbase, SUB)],
                             sem_st[bx])

    with jax.named_scope("p2"):
        for k in range(N_SUB):
            if k + NZ - 1 < N_SUB:
                issue_z(k + NZ - 1)
            if k + NX - 2 < N_SUB:
                issue_x(k + NX - 2)
            process(k)

    for k in range(max(0, N_SUB - NX), N_SUB):
        bx = k % NX
        nbase = gbase + k * SUB

        @pl.when(nbase < N_NODES)
        def _():
            pltpu.make_async_copy(
                xb[bx], out_hbm.at[pl.ds(nbase, SUB)], sem_st[bx]).wait()


def _flat_body(x_hbm, edges_hbm, zin_hbm, zout_hbm, out_hbm, *scratch):
    (ebuf0a, ebuf0b, ebuf1, hist_in, hist_out, shared, zsh, cbuf,
     idx_in, idx_out,
     xb0, xb1, xb2, xb3, zib0, zib1, zib2, zob0, zob1, zob2,
     sem_e0, sem_e0b, sem_e1, sem_c, sem_z,
     sx0, sx1, sx2, sx3, szi0, szi1, szi2, szo0, szo1, szo2,
     st0, st1, st2, st3) = scratch
    _body(x_hbm, edges_hbm, zin_hbm, zout_hbm, out_hbm,
          ebuf0a, ebuf0b, ebuf1, hist_in, hist_out, shared, zsh, cbuf,
          idx_in, idx_out,
          (xb0, xb1, xb2, xb3), (zib0, zib1, zib2), (zob0, zob1, zob2),
          sem_e0, sem_e0b, sem_e1, sem_c, sem_z,
          (sx0, sx1, sx2, sx3), (szi0, szi1, szi2), (szo0, szo1, szo2),
          (st0, st1, st2, st3))


@jax.jit
def _centrality(x, edge_index, z_in, z_out):
    mesh = plsc.VectorSubcoreMesh(core_axis_name="c", subcore_axis_name="s")
    run = functools.partial(
        pl.kernel,
        out_type=jax.ShapeDtypeStruct((N_NODES, NODE_DIM), jnp.float32),
        mesh=mesh,
        compiler_params=pltpu.CompilerParams(
            needs_layout_passes=False, skip_device_barrier=True),
        scratch_types=[
            pltpu.VMEM((2, ECH), jnp.int32),
            pltpu.VMEM((2, ECH), jnp.int32),
            pltpu.VMEM((2, EBLK), jnp.int32),
            pltpu.VMEM((NODES_PAD,), jnp.int32),
            pltpu.VMEM((NODES_PAD,), jnp.int32),
            pltpu.VMEM_SHARED((2 * NS * NODES_PAD,), jnp.int32),
            pltpu.VMEM_SHARED((2 * Z_ROWS, NODE_DIM), jnp.float32),
            pltpu.VMEM((NS * NODES_PER_TILE,), jnp.int32),
            pltpu.VMEM((NODES_PER_TILE,), jnp.int32),
            pltpu.VMEM((NODES_PER_TILE,), jnp.int32),
        ]
        + [pltpu.VMEM((SUB, NODE_DIM), jnp.float32)] * (NX + 2 * NZ)
        + [pltpu.SemaphoreType.DMA] * (5 + NX + 2 * NZ + NX),
    )(_flat_body)
    return run(x, edge_index, z_in, z_out)


def kernel(x, edge_index, z_in, z_out):
    return _centrality(x, edge_index.astype(jnp.int32), z_in, z_out)


# speedup vs baseline: 1.9685x; 1.0002x over previous
import functools

import jax
import jax.numpy as jnp
from jax import lax
from jax.experimental import pallas as pl
from jax.experimental.pallas import tpu as pltpu
from jax.experimental.pallas import tpu_sc as plsc

N_NODES = 10000
N_EDGES = 320000
NODE_DIM = 128
Z_ROWS = 256

NC = 2
NS = 16
L = 16

NODES_PAD = 10240
NODES_PER_TILE = NODES_PAD // (NC * NS)
SUB = 40
N_SUB = NODES_PER_TILE // SUB
NX = 4
NZ = 3
EBLK = 128
NBLK_BASE = 156
EDGES_BASE = NBLK_BASE * EBLK
ECHUNKS = 4
ECH_BLKS = NBLK_BASE // ECHUNKS
ECH = ECH_BLKS * EBLK
UNROLL = 8


def _body(x_hbm, edges_hbm, zin_hbm, zout_hbm, out_hbm,
          ebuf0a, ebuf0b, ebuf1, hist_in, hist_out, shared, zsh, cbuf,
          idx_in, idx_out, xb, zib, zob,
          sem_e0, sem_e0b, sem_e1, sem_c, sem_z,
          sem_x, sem_zi, sem_zo, sem_st):
    c = lax.axis_index("c")
    s = lax.axis_index("s")

    zeros = jnp.zeros((L,), jnp.int32)
    ones = jnp.ones((L,), jnp.int32)

    w = c * NS + s
    gbase = w * NODES_PER_TILE

    def issue_x(k):
        bx = k % NX
        nbase = gbase + k * SUB

        @pl.when(nbase < N_NODES)
        def _():
            if k >= NX:
                pltpu.make_async_copy(
                    xb[bx], out_hbm.at[pl.ds(gbase + (k - NX) * SUB, SUB)],
                    sem_st[bx]).wait()
            pltpu.async_copy(x_hbm.at[pl.ds(nbase, SUB)], xb[bx], sem_x[bx])

    def issue_z(k):
        bz = k % NZ
        nbase = gbase + k * SUB

        @pl.when(nbase < N_NODES)
        def _():
            pltpu.async_copy(
                zsh.at[idx_in.at[pl.ds(k * SUB, SUB)]], zib[bz], sem_zi[bz])
            pltpu.async_copy(
                zsh.at[idx_out.at[pl.ds(k * SUB, SUB)]], zob[bz], sem_zo[bz])

    @pl.when(s == 0)
    def _():
        pltpu.async_copy(zin_hbm, zsh.at[pl.ds(0, Z_ROWS)], sem_z)
        pltpu.async_copy(zout_hbm, zsh.at[pl.ds(Z_ROWS, Z_ROWS)], sem_z)

    ecol0 = (s * NBLK_BASE + jnp.minimum(s, 4)) * EBLK
    has_extra = s < 4
    eb = (ebuf0a, ebuf0b)
    sem_e = (sem_e0, sem_e0b)

    def issue_e(t):
        pltpu.async_copy(
            edges_hbm.at[:, pl.ds(ecol0 + t * ECH, ECH)], eb[t % 2],
            sem_e[t % 2])

    issue_e(0)
    issue_e(1)

    @pl.when(has_extra)
    def _():
        pltpu.async_copy(
            edges_hbm.at[:, pl.ds(ecol0 + EDGES_BASE, EBLK)], ebuf1, sem_e1)

    for k in range(NX - 2):
        issue_x(k)

    with jax.named_scope("p1_zero"):
        def zero_body(i, _):
            for u in range(8):
                hist_in[pl.ds((i * 8 + u) * L, L)] = zeros
                hist_out[pl.ds((i * 8 + u) * L, L)] = zeros
            return _

        lax.fori_loop(0, NODES_PAD // (8 * L), zero_body, None)

    def scatter_batch(ebuf, offs):
        srcs = [ebuf[0, pl.ds(o, L)] for o in offs]
        dsts = [ebuf[1, pl.ds(o, L)] for o in offs]
        for u in range(len(offs)):
            plsc.addupdate_scatter(hist_out, [srcs[u]], ones)
            plsc.addupdate_scatter(hist_in, [dsts[u]], ones)

    with jax.named_scope("p1_scatter"):
        for t in range(ECHUNKS):
            with jax.named_scope("p1_ewait"):
                pltpu.make_async_copy(
                    edges_hbm.at[:, pl.ds(ecol0 + t * ECH, ECH)], eb[t % 2],
                    sem_e[t % 2]).wait()
            def edge_body(i, _, _eb=eb[t % 2]):
                scatter_batch(
                    _eb, [(i * UNROLL + u) * L for u in range(UNROLL)])
                return _

            lax.fori_loop(0, ECH // (L * UNROLL), edge_body, None)
            if t + 2 < ECHUNKS:
                issue_e(t + 2)

        @pl.when(has_extra)
        def _():
            pltpu.make_async_copy(
                edges_hbm.at[:, pl.ds(ecol0 + EDGES_BASE, EBLK)], ebuf1,
                sem_e1).wait()
            scatter_batch(ebuf1, [u * L for u in range(EBLK // L)])

    @pl.when(s == 0)
    def _():
        pltpu.make_async_copy(
            zout_hbm, zsh.at[pl.ds(Z_ROWS, Z_ROWS)], sem_z).wait()
        pltpu.make_async_copy(
            zin_hbm, zsh.at[pl.ds(0, Z_ROWS)], sem_z).wait()

    with jax.named_scope("c_publish"):
        pltpu.sync_copy(hist_in, shared.at[pl.ds(s * NODES_PAD, NODES_PAD)])
        pltpu.sync_copy(
            hist_out, shared.at[pl.ds((NS + s) * NODES_PAD, NODES_PAD)])
    with jax.named_scope("c_barrier"):
        plsc.subcore_barrier()

    zmax = jnp.full((L,), Z_ROWS - 1, jnp.int32)

    def read_partials(which):
        cps = []
        for r in range(NS):
            cps.append(pltpu.async_copy(
                shared.at[pl.ds((which * NS + r) * NODES_PAD + gbase,
                                NODES_PER_TILE)],
                cbuf.at[pl.ds(r * NODES_PER_TILE, NODES_PER_TILE)],
                sem_c))
        return cps

    def combine(idx_ref, row_off):
        def comb_body(j, _):
            acc = cbuf[pl.ds(j * L, L)]
            for r in range(1, NS):
                acc = acc + cbuf[pl.ds(r * NODES_PER_TILE + j * L, L)]
            idx_ref[pl.ds(j * L, L)] = jnp.minimum(acc, zmax) + row_off
            return _

        lax.fori_loop(0, NODES_PER_TILE // L, comb_body, None)

    with jax.named_scope("c_read"):
        for cp in read_partials(0):
            cp.wait()
    with jax.named_scope("c_sum"):
        combine(idx_in, 0)
    with jax.named_scope("c_read2"):
        for cp in read_partials(1):
            cp.wait()
    with jax.named_scope("c_sum2"):
        combine(idx_out, Z_ROWS)

    with jax.named_scope("p2_zissue"):
        for k in range(NZ - 1):
            issue_z(k)

    def process(k):
        bx = k % NX
        bz = k % NZ
        nbase = gbase + k * SUB

        @pl.when(nbase < N_NODES)
        def _():
            with jax.named_scope("p2_waitx"):
                pltpu.make_async_copy(
                    x_hbm.at[pl.ds(nbase, SUB)], xb[bx], sem_x[bx]).wait()
            with jax.named_scope("p2_waitz"):
                pltpu.make_async_copy(
                    zsh.at[idx_in.at[pl.ds(k * SUB, SUB)]], zib[bz],
                    sem_zi[bz]).wait()
                pltpu.make_async_copy(
                    zsh.at[idx_out.at[pl.ds(k * SUB, SUB)]], zob[bz],
                    sem_zo[bz]).wait()

            def add_body(r, _):
                for cc in range(NODE_DIM // L):
                    sl = pl.ds(cc * L, L)
                    xb[bx][r, sl] = (
                        xb[bx][r, sl] + zib[bz][r, sl] + zob[bz][r, sl])
                return _

            with jax.named_scope("p2_add"):
                lax.fori_loop(0, SUB, add_body, None)
            pltpu.async_copy(xb[bx], out_hbm.at[pl.ds(nbase, SUB)],
                             sem_st[bx])

    with jax.named_scope("p2"):
        for k in range(N_SUB):
            if k + NZ - 1 < N_SUB:
                issue_z(k + NZ - 1)
            if k + NX - 2 < N_SUB:
                issue_x(k + NX - 2)
            process(k)

    for k in range(max(0, N_SUB - NX), N_SUB):
        bx = k % NX
        nbase = gbase + k * SUB

        @pl.when(nbase < N_NODES)
        def _():
            pltpu.make_async_copy(
                xb[bx], out_hbm.at[pl.ds(nbase, SUB)], sem_st[bx]).wait()


def _flat_body(x_hbm, edges_hbm, zin_hbm, zout_hbm, out_hbm, *scratch):
    (ebuf0a, ebuf0b, ebuf1, hist_in, hist_out, shared, zsh, cbuf,
     idx_in, idx_out,
     xb0, xb1, xb2, xb3, zib0, zib1, zib2, zob0, zob1, zob2,
     sem_e0, sem_e0b, sem_e1, sem_c, sem_z,
     sx0, sx1, sx2, sx3, szi0, szi1, szi2, szo0, szo1, szo2,
     st0, st1, st2, st3) = scratch
    _body(x_hbm, edges_hbm, zin_hbm, zout_hbm, out_hbm,
          ebuf0a, ebuf0b, ebuf1, hist_in, hist_out, shared, zsh, cbuf,
          idx_in, idx_out,
          (xb0, xb1, xb2, xb3), (zib0, zib1, zib2), (zob0, zob1, zob2),
          sem_e0, sem_e0b, sem_e1, sem_c, sem_z,
          (sx0, sx1, sx2, sx3), (szi0, szi1, szi2), (szo0, szo1, szo2),
          (st0, st1, st2, st3))


@jax.jit
def _centrality(x, edge_index, z_in, z_out):
    mesh = plsc.VectorSubcoreMesh(core_axis_name="c", subcore_axis_name="s")
    run = functools.partial(
        pl.kernel,
        out_type=jax.ShapeDtypeStruct((N_NODES, NODE_DIM), jnp.float32),
        mesh=mesh,
        compiler_params=pltpu.CompilerParams(needs_layout_passes=False),
        scratch_types=[
            pltpu.VMEM((2, ECH), jnp.int32),
            pltpu.VMEM((2, ECH), jnp.int32),
            pltpu.VMEM((2, EBLK), jnp.int32),
            pltpu.VMEM((NODES_PAD,), jnp.int32),
            pltpu.VMEM((NODES_PAD,), jnp.int32),
            pltpu.VMEM_SHARED((2 * NS * NODES_PAD,), jnp.int32),
            pltpu.VMEM_SHARED((2 * Z_ROWS, NODE_DIM), jnp.float32),
            pltpu.VMEM((NS * NODES_PER_TILE,), jnp.int32),
            pltpu.VMEM((NODES_PER_TILE,), jnp.int32),
            pltpu.VMEM((NODES_PER_TILE,), jnp.int32),
        ]
        + [pltpu.VMEM((SUB, NODE_DIM), jnp.float32)] * (NX + 2 * NZ)
        + [pltpu.SemaphoreType.DMA] * (5 + NX + 2 * NZ + NX),
    )(_flat_body)
    return run(x, edge_index, z_in, z_out)


def kernel(x, edge_index, z_in, z_out):
    return _centrality(x, edge_index.astype(jnp.int32), z_in, z_out)


# speedup vs baseline: 1.9725x; 1.0020x over previous
import functools

import jax
import jax.numpy as jnp
from jax import lax
from jax.experimental import pallas as pl
from jax.experimental.pallas import tpu as pltpu
from jax.experimental.pallas import tpu_sc as plsc

N_NODES = 10000
N_EDGES = 320000
NODE_DIM = 128
Z_ROWS = 256

NC = 2
NS = 16
L = 16

NODES_PAD = 10240
NODES_PER_TILE = NODES_PAD // (NC * NS)
SUB = 40
N_SUB = NODES_PER_TILE // SUB
NX = 4
NZ = 3
EBLK = 128
NBLK_BASE = 156
EDGES_BASE = NBLK_BASE * EBLK
ECHUNKS = 4
ECH_BLKS = NBLK_BASE // ECHUNKS
ECH = ECH_BLKS * EBLK
UNROLL = 8


def _body(x_hbm, edges_hbm, zin_hbm, zout_hbm, out_hbm,
          ebuf0a, ebuf0b, ebuf1, hist_in, hist_out, shared, zsh, cbuf,
          idx_in, idx_out, xb, zib, zob,
          sem_e0, sem_e0b, sem_e1, sem_c, sem_z,
          sem_x, sem_zi, sem_zo, sem_st):
    c = lax.axis_index("c")
    s = lax.axis_index("s")

    zeros = jnp.zeros((L,), jnp.int32)
    ones = jnp.ones((L,), jnp.int32)

    w = c * NS + s
    gbase = w * NODES_PER_TILE

    def issue_x(k):
        bx = k % NX
        nbase = gbase + k * SUB

        @pl.when(nbase < N_NODES)
        def _():
            if k >= NX:
                pltpu.make_async_copy(
                    xb[bx], out_hbm.at[pl.ds(gbase + (k - NX) * SUB, SUB)],
                    sem_st[bx]).wait()
            pltpu.async_copy(x_hbm.at[pl.ds(nbase, SUB)], xb[bx], sem_x[bx])

    def issue_z(k):
        bz = k % NZ
        nbase = gbase + k * SUB

        @pl.when(nbase < N_NODES)
        def _():
            pltpu.async_copy(
                zsh.at[idx_in.at[pl.ds(k * SUB, SUB)]], zib[bz], sem_zi[bz])
            pltpu.async_copy(
                zsh.at[idx_out.at[pl.ds(k * SUB, SUB)]], zob[bz], sem_zo[bz])

    @pl.when(s == 0)
    def _():
        pltpu.async_copy(zin_hbm, zsh.at[pl.ds(0, Z_ROWS)], sem_z)
        pltpu.async_copy(zout_hbm, zsh.at[pl.ds(Z_ROWS, Z_ROWS)], sem_z)

    ecol0 = (s * NBLK_BASE + jnp.minimum(s, 4)) * EBLK
    has_extra = s < 4
    eb = (ebuf0a, ebuf0b)
    sem_e = (sem_e0, sem_e0b)

    def issue_e(t):
        pltpu.async_copy(
            edges_hbm.at[:, pl.ds(ecol0 + t * ECH, ECH)], eb[t % 2],
            sem_e[t % 2])

    issue_e(0)
    issue_e(1)

    @pl.when(has_extra)
    def _():
        pltpu.async_copy(
            edges_hbm.at[:, pl.ds(ecol0 + EDGES_BASE, EBLK)], ebuf1, sem_e1)

    for k in range(NX - 2):
        issue_x(k)

    with jax.named_scope("p1_zero"):
        def zero_body(i, _):
            for u in range(8):
                hist_in[pl.ds((i * 8 + u) * L, L)] = zeros
                hist_out[pl.ds((i * 8 + u) * L, L)] = zeros
            return _

        lax.fori_loop(0, NODES_PAD // (8 * L), zero_body, None)

    def scatter_batch(ebuf, offs):
        srcs = [ebuf[0, pl.ds(o, L)] for o in offs]
        dsts = [ebuf[1, pl.ds(o, L)] for o in offs]
        for u in range(len(offs)):
            plsc.addupdate_scatter(hist_out, [srcs[u]], ones)
            plsc.addupdate_scatter(hist_in, [dsts[u]], ones)

    with jax.named_scope("p1_scatter"):
        for t in range(ECHUNKS):
            with jax.named_scope("p1_ewait"):
                pltpu.make_async_copy(
                    edges_hbm.at[:, pl.ds(ecol0 + t * ECH, ECH)], eb[t % 2],
                    sem_e[t % 2]).wait()
            @plsc.parallel_loop(0, ECH // (L * UNROLL), 1)
            def edge_body(i, _eb=eb[t % 2]):
                scatter_batch(
                    _eb, [(i * UNROLL + u) * L for u in range(UNROLL)])
            if t + 2 < ECHUNKS:
                issue_e(t + 2)

        @pl.when(has_extra)
        def _():
            pltpu.make_async_copy(
                edges_hbm.at[:, pl.ds(ecol0 + EDGES_BASE, EBLK)], ebuf1,
                sem_e1).wait()
            scatter_batch(ebuf1, [u * L for u in range(EBLK // L)])

    @pl.when(s == 0)
    def _():
        pltpu.make_async_copy(
            zout_hbm, zsh.at[pl.ds(Z_ROWS, Z_ROWS)], sem_z).wait()
        pltpu.make_async_copy(
            zin_hbm, zsh.at[pl.ds(0, Z_ROWS)], sem_z).wait()

    with jax.named_scope("c_publish"):
        pltpu.sync_copy(hist_in, shared.at[pl.ds(s * NODES_PAD, NODES_PAD)])
        pltpu.sync_copy(
            hist_out, shared.at[pl.ds((NS + s) * NODES_PAD, NODES_PAD)])
    with jax.named_scope("c_barrier"):
        plsc.subcore_barrier()

    zmax = jnp.full((L,), Z_ROWS - 1, jnp.int32)

    def read_partials(which):
        cps = []
        for r in range(NS):
            cps.append(pltpu.async_copy(
                shared.at[pl.ds((which * NS + r) * NODES_PAD + gbase,
                                NODES_PER_TILE)],
                cbuf.at[pl.ds(r * NODES_PER_TILE, NODES_PER_TILE)],
                sem_c))
        return cps

    def combine(idx_ref, row_off):
        def comb_body(j, _):
            acc = cbuf[pl.ds(j * L, L)]
            for r in range(1, NS):
                acc = acc + cbuf[pl.ds(r * NODES_PER_TILE + j * L, L)]
            idx_ref[pl.ds(j * L, L)] = jnp.minimum(acc, zmax) + row_off
            return _

        lax.fori_loop(0, NODES_PER_TILE // L, comb_body, None)

    with jax.named_scope("c_read"):
        for cp in read_partials(0):
            cp.wait()
    with jax.named_scope("c_sum"):
        combine(idx_in, 0)
    with jax.named_scope("c_read2"):
        for cp in read_partials(1):
            cp.wait()
    with jax.named_scope("c_sum2"):
        combine(idx_out, Z_ROWS)

    with jax.named_scope("p2_zissue"):
        for k in range(NZ - 1):
            issue_z(k)

    def process(k):
        bx = k % NX
        bz = k % NZ
        nbase = gbase + k * SUB

        @pl.when(nbase < N_NODES)
        def _():
            with jax.named_scope("p2_waitx"):
                pltpu.make_async_copy(
                    x_hbm.at[pl.ds(nbase, SUB)], xb[bx], sem_x[bx]).wait()
            with jax.named_scope("p2_waitz"):
                pltpu.make_async_copy(
                    zsh.at[idx_in.at[pl.ds(k * SUB, SUB)]], zib[bz],
                    sem_zi[bz]).wait()
                pltpu.make_async_copy(
                    zsh.at[idx_out.at[pl.ds(k * SUB, SUB)]], zob[bz],
                    sem_zo[bz]).wait()

            def add_body(r, _):
                for cc in range(NODE_DIM // L):
                    sl = pl.ds(cc * L, L)
                    xb[bx][r, sl] = (
                        xb[bx][r, sl] + zib[bz][r, sl] + zob[bz][r, sl])
                return _

            with jax.named_scope("p2_add"):
                lax.fori_loop(0, SUB, add_body, None)
            pltpu.async_copy(xb[bx], out_hbm.at[pl.ds(nbase, SUB)],
                             sem_st[bx])

    with jax.named_scope("p2"):
        for k in range(N_SUB):
            if k + NZ - 1 < N_SUB:
                issue_z(k + NZ - 1)
            if k + NX - 2 < N_SUB:
                issue_x(k + NX - 2)
            process(k)

    for k in range(max(0, N_SUB - NX), N_SUB):
        bx = k % NX
        nbase = gbase + k * SUB

        @pl.when(nbase < N_NODES)
        def _():
            pltpu.make_async_copy(
                xb[bx], out_hbm.at[pl.ds(nbase, SUB)], sem_st[bx]).wait()


def _flat_body(x_hbm, edges_hbm, zin_hbm, zout_hbm, out_hbm, *scratch):
    (ebuf0a, ebuf0b, ebuf1, hist_in, hist_out, shared, zsh, cbuf,
     idx_in, idx_out,
     xb0, xb1, xb2, xb3, zib0, zib1, zib2, zob0, zob1, zob2,
     sem_e0, sem_e0b, sem_e1, sem_c, sem_z,
     sx0, sx1, sx2, sx3, szi0, szi1, szi2, szo0, szo1, szo2,
     st0, st1, st2, st3) = scratch
    _body(x_hbm, edges_hbm, zin_hbm, zout_hbm, out_hbm,
          ebuf0a, ebuf0b, ebuf1, hist_in, hist_out, shared, zsh, cbuf,
          idx_in, idx_out,
          (xb0, xb1, xb2, xb3), (zib0, zib1, zib2), (zob0, zob1, zob2),
          sem_e0, sem_e0b, sem_e1, sem_c, sem_z,
          (sx0, sx1, sx2, sx3), (szi0, szi1, szi2), (szo0, szo1, szo2),
          (st0, st1, st2, st3))


@jax.jit
def _centrality(x, edge_index, z_in, z_out):
    mesh = plsc.VectorSubcoreMesh(core_axis_name="c", subcore_axis_name="s")
    run = functools.partial(
        pl.kernel,
        out_type=jax.ShapeDtypeStruct((N_NODES, NODE_DIM), jnp.float32),
        mesh=mesh,
        compiler_params=pltpu.CompilerParams(needs_layout_passes=False),
        scratch_types=[
            pltpu.VMEM((2, ECH), jnp.int32),
            pltpu.VMEM((2, ECH), jnp.int32),
            pltpu.VMEM((2, EBLK), jnp.int32),
            pltpu.VMEM((NODES_PAD,), jnp.int32),
            pltpu.VMEM((NODES_PAD,), jnp.int32),
            pltpu.VMEM_SHARED((2 * NS * NODES_PAD,), jnp.int32),
            pltpu.VMEM_SHARED((2 * Z_ROWS, NODE_DIM), jnp.float32),
            pltpu.VMEM((NS * NODES_PER_TILE,), jnp.int32),
            pltpu.VMEM((NODES_PER_TILE,), jnp.int32),
            pltpu.VMEM((NODES_PER_TILE,), jnp.int32),
        ]
        + [pltpu.VMEM((SUB, NODE_DIM), jnp.float32)] * (NX + 2 * NZ)
        + [pltpu.SemaphoreType.DMA] * (5 + NX + 2 * NZ + NX),
    )(_flat_body)
    return run(x, edge_index, z_in, z_out)


def kernel(x, edge_index, z_in, z_out):
    return _centrality(x, edge_index.astype(jnp.int32), z_in, z_out)
